# Initial kernel scaffold; baseline (speedup 1.0000x reference)
import jax
import jax.numpy as jnp
from jax.experimental import pallas as pl


def kernel(x, indices_layers_0, indices_layers_1):
    raise NotImplementedError("write your pallas kernel here")


# speedup vs baseline: 7.7519x; 7.7519x over previous
import functools

import jax
import jax.numpy as jnp
from jax import lax
from jax.experimental import pallas as pl
from jax.experimental.pallas import tpu as pltpu
from jax.experimental.pallas import tpu_sc as plsc

_NC, _NS = 2, 16
_NW = _NC * _NS
_E = 128
_C = 128
_G = _C // 4


def _make_level(M, N):
    nch = M // _C // _NW
    assert nch * _C * _NW == M
    mesh = plsc.VectorSubcoreMesh(
        core_axis_name="c", subcore_axis_name="s",
        num_cores=_NC, num_subcores=_NS)

    @functools.partial(
        pl.kernel,
        out_type=(jax.ShapeDtypeStruct((M, _E), jnp.float32),
                  jax.ShapeDtypeStruct((M // 4, _E), jnp.float32)),
        mesh=mesh,
        scratch_types=[
            pltpu.VMEM((nch, _C), jnp.int32),
            pltpu.VMEM((_C, _E), jnp.float32),
            pltpu.VMEM((_G, _E), jnp.float32),
            pltpu.SemaphoreType.DMA,
        ],
    )
    def level(table_hbm, idx_hbm, res_hbm, mean_hbm, idx_v, rows_v, mean_v, sem):
        wid = lax.axis_index("s") * _NC + lax.axis_index("c")
        pltpu.sync_copy(idx_hbm.at[pl.ds(wid * nch, nch)], idx_v)

        def chunk_body(j, carry):
            pltpu.async_copy(table_hbm.at[idx_v.at[j]], rows_v, sem).wait()

            def group_body(g, carry2):
                r = 4 * g
                for cb in range(_E // 16):
                    s = pl.ds(cb * 16, 16)
                    a = rows_v[r, s]
                    b = rows_v[r + 1, s]
                    c = rows_v[r + 2, s]
                    d = rows_v[r + 3, s]
                    m = ((a + b) + (c + d)) * 0.25
                    mean_v[g, s] = m
                    rows_v[r, s] = a - m
                    rows_v[r + 1, s] = b - m
                    rows_v[r + 2, s] = c - m
                    rows_v[r + 3, s] = d - m
                return carry2

            lax.fori_loop(0, _G, group_body, 0)
            row0 = (wid * nch + j) * _C
            grp0 = (wid * nch + j) * _G
            pltpu.sync_copy(rows_v, res_hbm.at[pl.ds(row0, _C)])
            pltpu.sync_copy(mean_v, mean_hbm.at[pl.ds(grp0, _G)])
            return carry

        lax.fori_loop(0, nch, chunk_body, 0)

    return level


def _level(table, idx):
    M = idx.shape[0]
    N = table.shape[0]
    idx2 = idx.reshape(M // _C, _C)
    return _make_level(M, N)(table, idx2)


def kernel(x, indices_layers_0, indices_layers_1):
    b, n, e = x.shape
    xf = x.reshape(b * n, e)
    offs0 = (jnp.arange(b, dtype=jnp.int32) * n)[:, None]
    idx0f = (indices_layers_0[None, :] + offs0).reshape(-1)
    r0, m0 = _level(xf, idx0f)
    n1 = n // 4
    offs1 = (jnp.arange(b, dtype=jnp.int32) * n1)[:, None]
    idx1f = (indices_layers_1[None, :] + offs1).reshape(-1)
    r1, m1 = _level(m0, idx1f)
    return (m1.reshape(b, n1 // 4, e),
            r1.reshape(b, n1, e),
            r0.reshape(b, n, e))


# speedup vs baseline: 10.6851x; 1.3784x over previous
import functools

import jax
import jax.numpy as jnp
from jax import lax
from jax.experimental import pallas as pl
from jax.experimental.pallas import tpu as pltpu
from jax.experimental.pallas import tpu_sc as plsc

_NC, _NS = 2, 16
_NW = _NC * _NS
_E = 128
_C = 128
_G = _C // 4


def _make_level(M, N):
    nch = M // _C // _NW
    assert nch * _C * _NW == M
    mesh = plsc.VectorSubcoreMesh(
        core_axis_name="c", subcore_axis_name="s",
        num_cores=_NC, num_subcores=_NS)

    @functools.partial(
        pl.kernel,
        out_type=(jax.ShapeDtypeStruct((M, _E), jnp.float32),
                  jax.ShapeDtypeStruct((M // 4, _E), jnp.float32)),
        mesh=mesh,
        scratch_types=[
            pltpu.VMEM((nch, _C), jnp.int32),
            pltpu.VMEM((2, _C, _E), jnp.float32),
            pltpu.VMEM((2, _G, _E), jnp.float32),
            pltpu.SemaphoreType.DMA((2,)),
            pltpu.SemaphoreType.DMA((2,)),
        ],
    )
    def level(table_hbm, idx_hbm, res_hbm, mean_hbm, idx_v, rows_v, mean_v,
              gsem, ssem):
        wid = lax.axis_index("s") * _NC + lax.axis_index("c")
        base = wid * nch
        pltpu.sync_copy(idx_hbm.at[pl.ds(base, nch)], idx_v)

        def compute(p):
            rows = rows_v.at[p]
            means = mean_v.at[p]

            def group_body(g, carry2):
                r = 4 * g
                for cb in range(_E // 16):
                    s = pl.ds(cb * 16, 16)
                    a = rows[r, s]
                    b = rows[r + 1, s]
                    c = rows[r + 2, s]
                    d = rows[r + 3, s]
                    m = ((a + b) + (c + d)) * 0.25
                    means[g, s] = m
                    rows[r, s] = a - m
                    rows[r + 1, s] = b - m
                    rows[r + 2, s] = c - m
                    rows[r + 3, s] = d - m
                return carry2

            lax.fori_loop(0, _G, group_body, 0)

        def store_descs(t, p):
            row0 = (base + t) * _C
            grp0 = (base + t) * _G
            return (
                pltpu.make_async_copy(rows_v.at[p],
                                      res_hbm.at[pl.ds(row0, _C)], ssem.at[p]),
                pltpu.make_async_copy(mean_v.at[p],
                                      mean_hbm.at[pl.ds(grp0, _G)], ssem.at[p]),
            )

        def gather_desc(t, p):
            return pltpu.make_async_copy(table_hbm.at[idx_v.at[t]],
                                         rows_v.at[p], gsem.at[p])

        gather_desc(0, 0).start()

        @pl.loop(0, nch, step=2)
        def chunk_pair(j):
            for p in range(2):
                t = j + p
                q = 1 - p

                @pl.when(t > 0)
                def _():
                    ra, rb = store_descs(t - 1, q)
                    ra.wait()
                    rb.wait()

                @pl.when(t + 1 < nch)
                def _():
                    gather_desc(t + 1, q).start()

                gather_desc(t, p).wait()
                compute(p)
                sa, sb = store_descs(t, p)
                sa.start()
                sb.start()

        ra, rb = store_descs(nch - 1, (nch - 1) % 2)
        ra.wait()
        rb.wait()

    return level


def _level(table, idx):
    M = idx.shape[0]
    N = table.shape[0]
    idx2 = idx.reshape(M // _C, _C)
    return _make_level(M, N)(table, idx2)


def kernel(x, indices_layers_0, indices_layers_1):
    b, n, e = x.shape
    xf = x.reshape(b * n, e)
    offs0 = (jnp.arange(b, dtype=jnp.int32) * n)[:, None]
    idx0f = (indices_layers_0[None, :] + offs0).reshape(-1)
    r0, m0 = _level(xf, idx0f)
    n1 = n // 4
    offs1 = (jnp.arange(b, dtype=jnp.int32) * n1)[:, None]
    idx1f = (indices_layers_1[None, :] + offs1).reshape(-1)
    r1, m1 = _level(m0, idx1f)
    return (m1.reshape(b, n1 // 4, e),
            r1.reshape(b, n1, e),
            r0.reshape(b, n, e))


# speedup vs baseline: 12.9537x; 1.2123x over previous
import functools

import jax
import jax.numpy as jnp
from jax import lax
from jax.experimental import pallas as pl
from jax.experimental.pallas import tpu as pltpu
from jax.experimental.pallas import tpu_sc as plsc

_NB = 4
_GA = 2
_NC, _NS = 2, 16
_NW = _NC * _NS
_E = 128
_C = 128
_G = _C // 4


def _make_level(M, N):
    nch = M // _C // _NW
    assert nch * _C * _NW == M
    mesh = plsc.VectorSubcoreMesh(
        core_axis_name="c", subcore_axis_name="s",
        num_cores=_NC, num_subcores=_NS)

    @functools.partial(
        pl.kernel,
        out_type=(jax.ShapeDtypeStruct((M, _E), jnp.float32),
                  jax.ShapeDtypeStruct((M // 4, _E), jnp.float32)),
        mesh=mesh,
        scratch_types=[
            pltpu.VMEM((nch, _C), jnp.int32),
            pltpu.VMEM((_NB, _C, _E), jnp.float32),
            pltpu.VMEM((_NB, _G, _E), jnp.float32),
            pltpu.SemaphoreType.DMA((_NB,)),
            pltpu.SemaphoreType.DMA((_NB,)),
        ],
    )
    def level(table_hbm, idx_hbm, res_hbm, mean_hbm, idx_v, rows_v, mean_v,
              gsem, ssem):
        wid = lax.axis_index("s") * _NC + lax.axis_index("c")
        base = wid * nch
        pltpu.sync_copy(idx_hbm.at[pl.ds(base, nch)], idx_v)

        def compute(p):
            rows = rows_v.at[p]
            means = mean_v.at[p]

            def group_body(g, carry2):
                r = 4 * g
                for cb in range(_E // 16):
                    s = pl.ds(cb * 16, 16)
                    a = rows[r, s]
                    b = rows[r + 1, s]
                    c = rows[r + 2, s]
                    d = rows[r + 3, s]
                    m = ((a + b) + (c + d)) * 0.25
                    means[g, s] = m
                    rows[r, s] = a - m
                    rows[r + 1, s] = b - m
                    rows[r + 2, s] = c - m
                    rows[r + 3, s] = d - m
                return carry2

            lax.fori_loop(0, _G, group_body, 0)

        def store_descs(t, p):
            row0 = (base + t) * _C
            grp0 = (base + t) * _G
            return (
                pltpu.make_async_copy(rows_v.at[p],
                                      res_hbm.at[pl.ds(row0, _C)], ssem.at[p]),
                pltpu.make_async_copy(mean_v.at[p],
                                      mean_hbm.at[pl.ds(grp0, _G)], ssem.at[p]),
            )

        def gather_desc(t, p):
            return pltpu.make_async_copy(table_hbm.at[idx_v.at[t]],
                                         rows_v.at[p], gsem.at[p])

        for t0 in range(_GA):
            gather_desc(t0, t0).start()

        @pl.loop(0, nch, step=_NB)
        def chunk_quad(j):
            for p in range(_NB):
                t = j + p
                q = (p + _GA) % _NB

                @pl.when(t + _GA - _NB >= 0)
                def _():
                    ra, rb = store_descs(t + _GA - _NB, q)
                    ra.wait()
                    rb.wait()

                @pl.when(t + _GA < nch)
                def _():
                    gather_desc(t + _GA, q).start()

                gather_desc(t, p).wait()
                compute(p)
                sa, sb = store_descs(t, p)
                sa.start()
                sb.start()

        for t0 in range(nch - _NB + _GA, nch):
            ra, rb = store_descs(t0, t0 % _NB)
            ra.wait()
            rb.wait()

    return level


def _level(table, idx):
    M = idx.shape[0]
    N = table.shape[0]
    idx2 = idx.reshape(M // _C, _C)
    return _make_level(M, N)(table, idx2)


def kernel(x, indices_layers_0, indices_layers_1):
    b, n, e = x.shape
    xf = x.reshape(b * n, e)
    offs0 = (jnp.arange(b, dtype=jnp.int32) * n)[:, None]
    idx0f = (indices_layers_0[None, :] + offs0).reshape(-1)
    r0, m0 = _level(xf, idx0f)
    n1 = n // 4
    offs1 = (jnp.arange(b, dtype=jnp.int32) * n1)[:, None]
    idx1f = (indices_layers_1[None, :] + offs1).reshape(-1)
    r1, m1 = _level(m0, idx1f)
    return (m1.reshape(b, n1 // 4, e),
            r1.reshape(b, n1, e),
            r0.reshape(b, n, e))
